# baseline (speedup 1.0000x reference)
import functools

import jax
import jax.numpy as jnp
from jax import lax
from jax.experimental import pallas as pl
from jax.experimental.pallas import tpu as pltpu
from jax.experimental.pallas import tpu_sc as plsc

N = 10000
N_PAD = 10112
E = 320000
D = 128
NC = 2
NS = 16
CK = 128
BI = 8
NB = 10
NCH_T = NB * BI
E_PAD = NC * NS * NCH_T * CK
RPT = N_PAD // NS

_mesh = plsc.VectorSubcoreMesh(core_axis_name="c", subcore_axis_name="s")


@functools.partial(
    pl.kernel,
    out_type=jax.ShapeDtypeStruct((NC, N_PAD, 16), jnp.float32),
    mesh=_mesh,
    scratch_types=[
        pltpu.VMEM((BI, CK), jnp.int32),
        pltpu.VMEM((BI, CK), jnp.int32),
        pltpu.VMEM((CK, 16), jnp.float32),
        pltpu.VMEM_SHARED((N_PAD, 16), jnp.float32),
        pltpu.SemaphoreType.DMA,
        pltpu.SemaphoreType.DMA,
        pltpu.SemaphoreType.DMA,
    ],
)
def _deg_count(dst_hbm, zeros16_hbm, out_hbm,
               didx0, didx1, ones_v, acc_sh, isem0, isem1, ssem):
    c = lax.axis_index("c")
    s = lax.axis_index("s")
    pltpu.sync_copy(zeros16_hbm.at[pl.ds(s * RPT, RPT)],
                    acc_sh.at[pl.ds(s * RPT, RPT)])
    pltpu.async_copy(dst_hbm.at[c, s, 0], didx0, isem0)
    for r in range(CK):
        ones_v[r] = jnp.ones((16,), jnp.float32)
    plsc.subcore_barrier()

    def do_block(b, di, dn, isem_this, isem_next, next_b):
        pltpu.make_async_copy(dst_hbm.at[c, s, b], di, isem_this).wait()
        if next_b is not None:
            pltpu.async_copy(dst_hbm.at[c, s, next_b], dn, isem_next)
        for j in range(BI):
            pltpu.async_copy(ones_v, acc_sh.at[di.at[j]], ssem, add=True)
        for j in range(BI):
            pltpu.make_async_copy(ones_v, acc_sh.at[di.at[j]], ssem).wait()

    def pair_body(t, carry):
        b = t * 2
        do_block(b, didx0, didx1, isem0, isem1, b + 1)
        do_block(b + 1, didx1, didx0, isem1, isem0, b + 2)
        return carry

    lax.fori_loop(0, (NB - 2) // 2, pair_body, 0)
    do_block(NB - 2, didx0, didx1, isem0, isem1, NB - 1)
    do_block(NB - 1, didx1, didx0, isem1, isem0, None)
    plsc.subcore_barrier()
    pltpu.sync_copy(acc_sh.at[pl.ds(s * RPT, RPT)],
                    out_hbm.at[c, pl.ds(s * RPT, RPT)])


@functools.partial(
    pl.kernel,
    out_type=jax.ShapeDtypeStruct((NC, N_PAD, D), jnp.float32),
    mesh=_mesh,
    scratch_types=[
        pltpu.VMEM((BI, CK), jnp.int32),
        pltpu.VMEM((BI, CK), jnp.int32),
        pltpu.VMEM((BI, CK), jnp.int32),
        pltpu.VMEM((BI, CK), jnp.int32),
        pltpu.VMEM((CK, D), jnp.float32),
        pltpu.VMEM((CK, D), jnp.float32),
        pltpu.VMEM_SHARED((N_PAD, D), jnp.float32),
        pltpu.SemaphoreType.DMA,
        pltpu.SemaphoreType.DMA,
        pltpu.SemaphoreType.DMA,
        pltpu.SemaphoreType.DMA,
        pltpu.SemaphoreType.DMA,
        pltpu.SemaphoreType.DMA,
    ],
)
def _edge_agg(g_hbm, src_hbm, dst_hbm, zeros_hbm, out_hbm,
              sidx0, sidx1, didx0, didx1, rows0, rows1, acc_sh,
              isem0, isem1, gsem0, gsem1, ssem0, ssem1):
    c = lax.axis_index("c")
    s = lax.axis_index("s")
    pltpu.sync_copy(zeros_hbm.at[pl.ds(s * RPT, RPT)],
                    acc_sh.at[pl.ds(s * RPT, RPT)])
    pltpu.async_copy(src_hbm.at[c, s, 0], sidx0, isem0)
    pltpu.async_copy(dst_hbm.at[c, s, 0], didx0, isem0)
    plsc.subcore_barrier()

    def do_block(b, si, di, sn, dn, isem_this, isem_next, next_b):
        pltpu.make_async_copy(src_hbm.at[c, s, b], si, isem_this).wait()
        pltpu.make_async_copy(dst_hbm.at[c, s, b], di, isem_this).wait()
        if next_b is not None:
            pltpu.async_copy(src_hbm.at[c, s, next_b], sn, isem_next)
            pltpu.async_copy(dst_hbm.at[c, s, next_b], dn, isem_next)

        for j in range(BI):
            pltpu.async_copy(g_hbm.at[si.at[j]], rows0, gsem0).wait()
            pltpu.async_copy(rows0, acc_sh.at[di.at[j]], ssem0,
                             add=True).wait()

    def pair_body(t, carry):
        b = t * 2
        do_block(b, sidx0, didx0, sidx1, didx1, isem0, isem1, b + 1)
        do_block(b + 1, sidx1, didx1, sidx0, didx0, isem1, isem0, b + 2)
        return carry

    lax.fori_loop(0, (NB - 2) // 2, pair_body, 0)
    do_block(NB - 2, sidx0, didx0, sidx1, didx1, isem0, isem1, NB - 1)
    do_block(NB - 1, sidx1, didx1, sidx0, didx0, isem1, isem0, None)
    plsc.subcore_barrier()
    pltpu.sync_copy(acc_sh.at[pl.ds(s * RPT, RPT)],
                    out_hbm.at[c, pl.ds(s * RPT, RPT)])


BLK = 1000
GRID = N // BLK

_row_spec = pl.BlockSpec((BLK, D), lambda i: (i, 0))
_w_spec = pl.BlockSpec((D, D), lambda i: (0, 0))
_b_spec = pl.BlockSpec((1, D), lambda i: (0, 0))
_p_spec = pl.BlockSpec((NC, BLK, D), lambda i: (0, i, 0))
_da_spec = pl.BlockSpec((1, BLK, 1), lambda i: (0, i, 0))
_db_spec = pl.BlockSpec((1, BLK, 1), lambda i: (1, i, 0))


def _dis_from(da_ref, db_ref):
    return lax.rsqrt(da_ref[0] + db_ref[0] + 2.0)


def _mm_scale_body(x_ref, w_ref, da_ref, db_ref, h_ref, g_ref):
    h = jnp.dot(x_ref[...], w_ref[...], preferred_element_type=jnp.float32)
    dis = _dis_from(da_ref, db_ref)
    h_ref[...] = h
    g_ref[...] = h * dis


def _layer1_mm(x, W1, degp):
    return pl.pallas_call(
        _mm_scale_body,
        grid=(GRID,),
        in_specs=[_row_spec, _w_spec, _da_spec, _db_spec],
        out_specs=[_row_spec, _row_spec],
        out_shape=[jax.ShapeDtypeStruct((N, D), jnp.float32),
                   jax.ShapeDtypeStruct((N, D), jnp.float32)],
    )(x, W1, degp, degp)


def _combine_mm_body(p_ref, hm_ref, da_ref, db_ref, b_ref, w_ref,
                     h1_ref, hm2_ref, g2_ref):
    dis = _dis_from(da_ref, db_ref)
    acc = p_ref[0] + p_ref[1]
    h1 = jnp.maximum(
        acc * dis + hm_ref[...] * (2.0 * dis * dis) + b_ref[...], 0.0)
    hm2 = jnp.dot(h1, w_ref[...], preferred_element_type=jnp.float32)
    h1_ref[...] = h1
    hm2_ref[...] = hm2
    g2_ref[...] = hm2 * dis


def _layer2_mm(p1, h1m, degp, b1, W2):
    return pl.pallas_call(
        _combine_mm_body,
        grid=(GRID,),
        in_specs=[_p_spec, _row_spec, _da_spec, _db_spec, _b_spec, _w_spec],
        out_specs=[_row_spec, _row_spec, _row_spec],
        out_shape=[jax.ShapeDtypeStruct((N, D), jnp.float32),
                   jax.ShapeDtypeStruct((N, D), jnp.float32),
                   jax.ShapeDtypeStruct((N, D), jnp.float32)],
    )(p1, h1m, degp, degp, b1, W2)


def _final_body(p_ref, hm_ref, da_ref, db_ref, b_ref, out_ref):
    dis = _dis_from(da_ref, db_ref)
    acc = p_ref[0] + p_ref[1]
    out_ref[...] = acc * dis + hm_ref[...] * (2.0 * dis * dis) + b_ref[...]


def _final_combine(p2, h2m, degp, b2):
    return pl.pallas_call(
        _final_body,
        grid=(GRID,),
        in_specs=[_p_spec, _row_spec, _da_spec, _db_spec, _b_spec],
        out_specs=_row_spec,
        out_shape=jax.ShapeDtypeStruct((N, D), jnp.float32),
    )(p2, h2m, degp, degp, b2)


def kernel(x, edge_index, W1, b1, W2, b2):
    pad_src = jnp.zeros((E_PAD - E,), jnp.int32)
    pad_dst = jnp.full((E_PAD - E,), N, jnp.int32)
    srcp = jnp.concatenate([edge_index[0], pad_src]).reshape(
        NC, NS, NB, BI, CK)
    dstp = jnp.concatenate([edge_index[1], pad_dst]).reshape(
        NC, NS, NB, BI, CK)
    zeros128 = jnp.zeros((N_PAD, D), jnp.float32)
    zeros16 = jnp.zeros((N_PAD, 16), jnp.float32)

    degp = _deg_count(dstp, zeros16)
    degcol = degp[:, :, :1]

    h1m, g1 = _layer1_mm(x, W1, degcol)
    p1 = _edge_agg(g1, srcp, dstp, zeros128)
    h1, h2m, g2 = _layer2_mm(p1, h1m, degcol, b1.reshape(1, D), W2)
    p2 = _edge_agg(g2, srcp, dstp, zeros128)
    out = _final_combine(p2, h2m, degcol, b2.reshape(1, D))
    return (out, h1)


# speedup vs baseline: 2.2361x; 2.2361x over previous
import functools

import jax
import jax.numpy as jnp
from jax import lax
from jax.experimental import pallas as pl
from jax.experimental.pallas import tpu as pltpu
from jax.experimental.pallas import tpu_sc as plsc

N = 10000
N_PAD = 10112
E = 320000
D = 128
NC = 2
NS = 16
CK = 128
BI = 8
NB = 10
NCH_T = NB * BI
E_PAD = NC * NS * NCH_T * CK

ACK = 80
ANCH = E // (NC * NS * ACK)
RPT = N_PAD // NS

_mesh = plsc.VectorSubcoreMesh(core_axis_name="c", subcore_axis_name="s")


@functools.partial(
    pl.kernel,
    out_type=jax.ShapeDtypeStruct((NC, N_PAD, 16), jnp.float32),
    mesh=_mesh,
    scratch_types=[
        pltpu.VMEM((BI, CK), jnp.int32),
        pltpu.VMEM((BI, CK), jnp.int32),
        pltpu.VMEM((CK, 16), jnp.float32),
        pltpu.VMEM_SHARED((N_PAD, 16), jnp.float32),
        pltpu.SemaphoreType.DMA,
        pltpu.SemaphoreType.DMA,
        pltpu.SemaphoreType.DMA,
    ],
)
def _deg_count(dst_hbm, zeros16_hbm, out_hbm,
               didx0, didx1, ones_v, acc_sh, isem0, isem1, ssem):
    c = lax.axis_index("c")
    s = lax.axis_index("s")
    pltpu.sync_copy(zeros16_hbm.at[pl.ds(s * RPT, RPT)],
                    acc_sh.at[pl.ds(s * RPT, RPT)])
    pltpu.async_copy(dst_hbm.at[c, s, 0], didx0, isem0)
    for r in range(CK):
        ones_v[r] = jnp.ones((16,), jnp.float32)
    plsc.subcore_barrier()

    def do_block(b, di, dn, isem_this, isem_next, next_b):
        pltpu.make_async_copy(dst_hbm.at[c, s, b], di, isem_this).wait()
        if next_b is not None:
            pltpu.async_copy(dst_hbm.at[c, s, next_b], dn, isem_next)
        for j in range(BI):
            pltpu.async_copy(ones_v, acc_sh.at[di.at[j]], ssem, add=True)
        for j in range(BI):
            pltpu.make_async_copy(ones_v, acc_sh.at[di.at[j]], ssem).wait()

    def pair_body(t, carry):
        b = t * 2
        do_block(b, didx0, didx1, isem0, isem1, b + 1)
        do_block(b + 1, didx1, didx0, isem1, isem0, b + 2)
        return carry

    lax.fori_loop(0, (NB - 2) // 2, pair_body, 0)
    do_block(NB - 2, didx0, didx1, isem0, isem1, NB - 1)
    do_block(NB - 1, didx1, didx0, isem1, isem0, None)
    plsc.subcore_barrier()
    pltpu.sync_copy(acc_sh.at[pl.ds(s * RPT, RPT)],
                    out_hbm.at[c, pl.ds(s * RPT, RPT)])


@functools.partial(
    pl.kernel,
    out_type=jax.ShapeDtypeStruct((NC, N_PAD, D), jnp.float32),
    mesh=_mesh,
    scratch_types=[
        pltpu.VMEM((ANCH, ACK), jnp.int32),
        pltpu.VMEM((ANCH, ACK), jnp.int32),
        pltpu.VMEM((ACK, D), jnp.float32),
        pltpu.VMEM_SHARED((N_PAD, D), jnp.float32),
        pltpu.SemaphoreType.DMA,
        pltpu.SemaphoreType.DMA,
    ],
)
def _edge_agg(g_hbm, src_hbm, dst_hbm, zeros_hbm, out_hbm,
              src_v, dst_v, rows0, acc_sh, gsem, ssem):
    c = lax.axis_index("c")
    s = lax.axis_index("s")
    pltpu.sync_copy(zeros_hbm.at[pl.ds(s * RPT, RPT)],
                    acc_sh.at[pl.ds(s * RPT, RPT)])
    pltpu.sync_copy(src_hbm.at[c, s], src_v)
    pltpu.sync_copy(dst_hbm.at[c, s], dst_v)
    plsc.subcore_barrier()

    def chunk_body(i, carry):
        pltpu.async_copy(g_hbm.at[src_v.at[i]], rows0, gsem).wait()
        pltpu.async_copy(rows0, acc_sh.at[dst_v.at[i]], ssem,
                         add=True).wait()
        return carry

    lax.fori_loop(0, ANCH, chunk_body, 0)
    plsc.subcore_barrier()
    pltpu.sync_copy(acc_sh.at[pl.ds(s * RPT, RPT)],
                    out_hbm.at[c, pl.ds(s * RPT, RPT)])


BLK = 1000
GRID = N // BLK

_row_spec = pl.BlockSpec((BLK, D), lambda i: (i, 0))
_w_spec = pl.BlockSpec((D, D), lambda i: (0, 0))
_b_spec = pl.BlockSpec((1, D), lambda i: (0, 0))
_p_spec = pl.BlockSpec((NC, BLK, D), lambda i: (0, i, 0))
_da_spec = pl.BlockSpec((1, BLK, 1), lambda i: (0, i, 0))
_db_spec = pl.BlockSpec((1, BLK, 1), lambda i: (1, i, 0))


def _dis_from(da_ref, db_ref):
    return lax.rsqrt(da_ref[0] + db_ref[0] + 2.0)


def _mm_scale_body(x_ref, w_ref, da_ref, db_ref, h_ref, g_ref):
    h = jnp.dot(x_ref[...], w_ref[...], preferred_element_type=jnp.float32)
    dis = _dis_from(da_ref, db_ref)
    h_ref[...] = h
    g_ref[...] = h * dis


def _layer1_mm(x, W1, degp):
    return pl.pallas_call(
        _mm_scale_body,
        grid=(GRID,),
        in_specs=[_row_spec, _w_spec, _da_spec, _db_spec],
        out_specs=[_row_spec, _row_spec],
        out_shape=[jax.ShapeDtypeStruct((N, D), jnp.float32),
                   jax.ShapeDtypeStruct((N, D), jnp.float32)],
    )(x, W1, degp, degp)


def _combine_mm_body(p_ref, hm_ref, da_ref, db_ref, b_ref, w_ref,
                     h1_ref, hm2_ref, g2_ref):
    dis = _dis_from(da_ref, db_ref)
    acc = p_ref[0] + p_ref[1]
    h1 = jnp.maximum(
        acc * dis + hm_ref[...] * (2.0 * dis * dis) + b_ref[...], 0.0)
    hm2 = jnp.dot(h1, w_ref[...], preferred_element_type=jnp.float32)
    h1_ref[...] = h1
    hm2_ref[...] = hm2
    g2_ref[...] = hm2 * dis


def _layer2_mm(p1, h1m, degp, b1, W2):
    return pl.pallas_call(
        _combine_mm_body,
        grid=(GRID,),
        in_specs=[_p_spec, _row_spec, _da_spec, _db_spec, _b_spec, _w_spec],
        out_specs=[_row_spec, _row_spec, _row_spec],
        out_shape=[jax.ShapeDtypeStruct((N, D), jnp.float32),
                   jax.ShapeDtypeStruct((N, D), jnp.float32),
                   jax.ShapeDtypeStruct((N, D), jnp.float32)],
    )(p1, h1m, degp, degp, b1, W2)


def _final_body(p_ref, hm_ref, da_ref, db_ref, b_ref, out_ref):
    dis = _dis_from(da_ref, db_ref)
    acc = p_ref[0] + p_ref[1]
    out_ref[...] = acc * dis + hm_ref[...] * (2.0 * dis * dis) + b_ref[...]


def _final_combine(p2, h2m, degp, b2):
    return pl.pallas_call(
        _final_body,
        grid=(GRID,),
        in_specs=[_p_spec, _row_spec, _da_spec, _db_spec, _b_spec],
        out_specs=_row_spec,
        out_shape=jax.ShapeDtypeStruct((N, D), jnp.float32),
    )(p2, h2m, degp, degp, b2)


def kernel(x, edge_index, W1, b1, W2, b2):
    pad_dst = jnp.full((E_PAD - E,), N, jnp.int32)
    dstp = jnp.concatenate([edge_index[1], pad_dst]).reshape(
        NC, NS, NB, BI, CK)
    srca = edge_index[0].reshape(NC, NS, ANCH, ACK)
    dsta = edge_index[1].reshape(NC, NS, ANCH, ACK)
    zeros128 = jnp.zeros((N_PAD, D), jnp.float32)
    zeros16 = jnp.zeros((N_PAD, 16), jnp.float32)

    degp = _deg_count(dstp, zeros16)
    degcol = degp[:, :, :1]

    h1m, g1 = _layer1_mm(x, W1, degcol)
    p1 = _edge_agg(g1, srca, dsta, zeros128)
    h1, h2m, g2 = _layer2_mm(p1, h1m, degcol, b1.reshape(1, D), W2)
    p2 = _edge_agg(g2, srca, dsta, zeros128)
    out = _final_combine(p2, h2m, degcol, b2.reshape(1, D))
    return (out, h1)
